# Initial kernel scaffold; baseline (speedup 1.0000x reference)
import jax
import jax.numpy as jnp
from jax.experimental import pallas as pl


def kernel(x, hh_indices, hh_values, hh_bias, W_ih):
    raise NotImplementedError("write your pallas kernel here")


# speedup vs baseline: 15.0893x; 15.0893x over previous
import functools

import jax
import jax.numpy as jnp
from jax import lax
from jax.experimental import pallas as pl
from jax.experimental.pallas import tpu as pltpu
from jax.experimental.pallas import tpu_sc as plsc

_NSUB = 16
_LANE = 16
_CHUNK = 128
_CG = 16


def _ih_matmul(x2, w_pad, b_pad, n_tile):
    tb, k = x2.shape
    hp = w_pad.shape[0]
    grid = (hp // n_tile,)

    def body(x_ref, w_ref, b_ref, o_ref):
        acc = lax.dot_general(
            x_ref[...], w_ref[...],
            (((1,), (1,)), ((), ())),
            preferred_element_type=jnp.float32,
        )
        o_ref[...] = acc + b_ref[...]

    return pl.pallas_call(
        body,
        grid=grid,
        in_specs=[
            pl.BlockSpec((tb, k), lambda j: (0, 0)),
            pl.BlockSpec((n_tile, k), lambda j: (j, 0)),
            pl.BlockSpec((1, n_tile), lambda j: (0, j)),
        ],
        out_specs=pl.BlockSpec((tb, n_tile), lambda j: (0, j)),
        out_shape=jax.ShapeDtypeStruct((tb, hp), jnp.float32),
    )(x2, w_pad, b_pad)


def _make_sc_step(h_dim, rows_per_sub, n_chunks):
    mesh = plsc.VectorSubcoreMesh(
        core_axis_name="c", subcore_axis_name="s", num_cores=1)
    n_groups = n_chunks // _CG

    def step_body(h_ref, ih_ref, cols_ref, rows_ref, vals_ref, zeros_ref,
                  out_ref, acc, colsv, rowsv, valsv, gath, combv, ihv,
                  gsem, ssem):
        w = lax.axis_index("s")
        slab = pl.ds(w * rows_per_sub, rows_per_sub)

        pltpu.sync_copy(zeros_ref.at[slab], acc.at[slab])
        plsc.subcore_barrier()

        def group(g, _):
            base = g * _CG
            pltpu.sync_copy(cols_ref.at[w, pl.ds(base, _CG)], colsv)
            pltpu.sync_copy(rows_ref.at[w, pl.ds(base, _CG)], rowsv)
            pltpu.sync_copy(vals_ref.at[w, pl.ds(base, _CG)], valsv)
            copies = []
            for j in range(_CG):
                copies.append(pltpu.async_copy(
                    h_ref.at[colsv.at[j]], gath.at[j], gsem))
            for c in copies:
                c.wait()
            scatters = []
            for j in range(_CG):
                def mul16(i, _):
                    e0 = i * _LANE
                    vv = valsv[j, pl.ds(e0, _LANE)]
                    for u in range(_LANE):
                        gath[j, e0 + u, :] = gath[j, e0 + u, :] * vv[u]
                    return 0
                lax.fori_loop(0, _CHUNK // _LANE, mul16, 0)
                scatters.append(pltpu.async_copy(
                    gath.at[j], acc.at[rowsv.at[j]], ssem, add=True))
            for c in scatters:
                c.wait()
            return 0

        lax.fori_loop(0, n_groups, group, 0)
        plsc.subcore_barrier()

        pltpu.sync_copy(acc.at[slab], combv)
        pltpu.sync_copy(ih_ref.at[slab], ihv)

        def comb(i, _):
            a = combv[i, :] + ihv[i, :]
            e2 = jnp.exp(a * 2.0)
            combv[i, :] = 1.0 - 2.0 / (e2 + 1.0)
            return 0

        lax.fori_loop(0, rows_per_sub, comb, 0, unroll=4)
        pltpu.sync_copy(combv, out_ref.at[slab])

    return pl.kernel(
        step_body,
        out_type=jax.ShapeDtypeStruct((h_dim, _LANE), jnp.float32),
        mesh=mesh,
        scratch_types=[
            pltpu.VMEM_SHARED((h_dim, _LANE), jnp.float32),
            pltpu.VMEM((_CG, _CHUNK), jnp.int32),
            pltpu.VMEM((_CG, _CHUNK), jnp.int32),
            pltpu.VMEM((_CG, _CHUNK), jnp.float32),
            pltpu.VMEM((_CG, _CHUNK, _LANE), jnp.float32),
            pltpu.VMEM((rows_per_sub, _LANE), jnp.float32),
            pltpu.VMEM((rows_per_sub, _LANE), jnp.float32),
            pltpu.SemaphoreType.DMA,
            pltpu.SemaphoreType.DMA,
        ],
        compiler_params=pltpu.CompilerParams(use_tc_tiling_on_sc=False),
    )


@jax.jit
def kernel(x, hh_indices, hh_values, hh_bias, W_ih):
    b, t, d_in = x.shape
    h_dim = W_ih.shape[0]
    nnz = hh_values.shape[0]
    assert b == _LANE

    n_tile = 2048
    hp = ((h_dim + n_tile - 1) // n_tile) * n_tile
    w_pad = jnp.pad(W_ih, ((0, hp - h_dim), (0, 0)))
    b_pad = jnp.pad(hh_bias.reshape(1, h_dim), ((0, 0), (0, hp - h_dim)))
    x2 = x.swapaxes(0, 1).reshape(t * b, d_in)
    ih2 = _ih_matmul(x2, w_pad, b_pad, n_tile)
    ihs = ih2.reshape(t, b, hp).transpose(0, 2, 1)

    stride = _CHUNK * _CG
    epw = ((nnz // _NSUB + stride - 1) // stride) * stride
    ep = epw * _NSUB
    rows = hh_indices[0]
    cols = hh_indices[1]
    cols_p = jnp.pad(cols, (0, ep - nnz)).reshape(_NSUB, epw // _CHUNK, _CHUNK)
    rows_p = jnp.pad(rows, (0, ep - nnz)).reshape(_NSUB, epw // _CHUNK, _CHUNK)
    vals_p = jnp.pad(hh_values, (0, ep - nnz)).reshape(
        _NSUB, epw // _CHUNK, _CHUNK)

    zeros = jnp.zeros((hp, _LANE), jnp.float32)
    sc_step = _make_sc_step(hp, hp // _NSUB, epw // _CHUNK)

    def step(h, ih_t):
        h_new = sc_step(h, ih_t, cols_p, rows_p, vals_p, zeros)
        return h_new, h_new

    _, ys = lax.scan(step, zeros, ihs)
    return ys[:, :h_dim, :].transpose(2, 0, 1)


# speedup vs baseline: 17.6595x; 1.1703x over previous
import functools

import jax
import jax.numpy as jnp
from jax import lax
from jax.experimental import pallas as pl
from jax.experimental.pallas import tpu as pltpu
from jax.experimental.pallas import tpu_sc as plsc

_NSUB = 16
_LANE = 16
_CHUNK = 128
_CG = 16


def _ih_matmul(x2, w_pad, b_pad, n_tile):
    tb, k = x2.shape
    hp = w_pad.shape[0]
    grid = (hp // n_tile,)

    def body(x_ref, w_ref, b_ref, o_ref):
        acc = lax.dot_general(
            x_ref[...], w_ref[...],
            (((1,), (1,)), ((), ())),
            preferred_element_type=jnp.float32,
        )
        o_ref[...] = acc + b_ref[...]

    return pl.pallas_call(
        body,
        grid=grid,
        in_specs=[
            pl.BlockSpec((tb, k), lambda j: (0, 0)),
            pl.BlockSpec((n_tile, k), lambda j: (j, 0)),
            pl.BlockSpec((1, n_tile), lambda j: (0, j)),
        ],
        out_specs=pl.BlockSpec((tb, n_tile), lambda j: (0, j)),
        out_shape=jax.ShapeDtypeStruct((tb, hp), jnp.float32),
    )(x2, w_pad, b_pad)


def _make_sc_rnn(hp, rows_per_sub, n_chunks, t_steps):
    mesh = plsc.VectorSubcoreMesh(
        core_axis_name="c", subcore_axis_name="s", num_cores=1)
    n_groups = n_chunks // _CG

    def body(ih_ref, cols_ref, rows_ref, vals_ref, zeros_ref,
             ys_ref, h_ref,
             acc, colsv, rowsv, valsv, gath, combv, ihv, gsem, ssem):
        w = lax.axis_index("s")
        slab = pl.ds(w * rows_per_sub, rows_per_sub)

        pltpu.sync_copy(zeros_ref.at[slab], acc.at[slab])
        plsc.subcore_barrier()

        def step(t, _):
            @pl.when(t > 0)
            def phase1():
                def group(g, _):
                    base = g * _CG
                    pltpu.sync_copy(cols_ref.at[w, pl.ds(base, _CG)], colsv)
                    pltpu.sync_copy(rows_ref.at[w, pl.ds(base, _CG)], rowsv)
                    pltpu.sync_copy(vals_ref.at[w, pl.ds(base, _CG)], valsv)
                    copies = []
                    for j in range(_CG):
                        copies.append(pltpu.async_copy(
                            h_ref.at[colsv.at[j]], gath.at[j], gsem))
                    for c in copies:
                        c.wait()
                    scatters = []
                    for j in range(_CG):
                        def mul16(i, _):
                            e0 = i * _LANE
                            vv = valsv[j, pl.ds(e0, _LANE)]
                            for u in range(_LANE):
                                gath[j, e0 + u, :] = gath[j, e0 + u, :] * vv[u]
                            return 0
                        lax.fori_loop(0, _CHUNK // _LANE, mul16, 0)
                        scatters.append(pltpu.async_copy(
                            gath.at[j], acc.at[rowsv.at[j]], ssem, add=True))
                    for c in scatters:
                        c.wait()
                    return 0

                lax.fori_loop(0, n_groups, group, 0)

            plsc.subcore_barrier()

            pltpu.sync_copy(acc.at[slab], combv)
            pltpu.sync_copy(zeros_ref.at[slab], acc.at[slab])
            pltpu.sync_copy(ih_ref.at[t, slab], ihv)

            def comb(i, _):
                a = combv[i, :] + ihv[i, :]
                e2 = jnp.exp(a * 2.0)
                combv[i, :] = 1.0 - 2.0 / (e2 + 1.0)
                return 0

            lax.fori_loop(0, rows_per_sub, comb, 0, unroll=4)
            pltpu.sync_copy(combv, h_ref.at[slab])
            pltpu.sync_copy(combv, ys_ref.at[t, slab])
            plsc.subcore_barrier()
            return 0

        lax.fori_loop(0, t_steps, step, 0)

    return pl.kernel(
        body,
        out_type=(
            jax.ShapeDtypeStruct((t_steps, hp, _LANE), jnp.float32),
            jax.ShapeDtypeStruct((hp, _LANE), jnp.float32),
        ),
        mesh=mesh,
        scratch_types=[
            pltpu.VMEM_SHARED((hp, _LANE), jnp.float32),
            pltpu.VMEM((_CG, _CHUNK), jnp.int32),
            pltpu.VMEM((_CG, _CHUNK), jnp.int32),
            pltpu.VMEM((_CG, _CHUNK), jnp.float32),
            pltpu.VMEM((_CG, _CHUNK, _LANE), jnp.float32),
            pltpu.VMEM((rows_per_sub, _LANE), jnp.float32),
            pltpu.VMEM((rows_per_sub, _LANE), jnp.float32),
            pltpu.SemaphoreType.DMA,
            pltpu.SemaphoreType.DMA,
        ],
        compiler_params=pltpu.CompilerParams(use_tc_tiling_on_sc=False),
    )


@jax.jit
def kernel(x, hh_indices, hh_values, hh_bias, W_ih):
    b, t, d_in = x.shape
    h_dim = W_ih.shape[0]
    nnz = hh_values.shape[0]
    assert b == _LANE

    n_tile = 2048
    hp = ((h_dim + n_tile - 1) // n_tile) * n_tile
    w_pad = jnp.pad(W_ih, ((0, hp - h_dim), (0, 0)))
    b_pad = jnp.pad(hh_bias.reshape(1, h_dim), ((0, 0), (0, hp - h_dim)))
    x2 = x.swapaxes(0, 1).reshape(t * b, d_in)
    ih2 = _ih_matmul(x2, w_pad, b_pad, n_tile)
    ihs = ih2.reshape(t, b, hp).transpose(0, 2, 1)

    stride = _CHUNK * _CG
    epw = ((nnz // _NSUB + stride - 1) // stride) * stride
    ep = epw * _NSUB
    rows = hh_indices[0]
    cols = hh_indices[1]
    cols_p = jnp.pad(cols, (0, ep - nnz)).reshape(_NSUB, epw // _CHUNK, _CHUNK)
    rows_p = jnp.pad(rows, (0, ep - nnz)).reshape(_NSUB, epw // _CHUNK, _CHUNK)
    vals_p = jnp.pad(hh_values, (0, ep - nnz)).reshape(
        _NSUB, epw // _CHUNK, _CHUNK)

    zeros = jnp.zeros((hp, _LANE), jnp.float32)
    sc_rnn = _make_sc_rnn(hp, hp // _NSUB, epw // _CHUNK, t)
    ys, _ = sc_rnn(ihs, cols_p, rows_p, vals_p, zeros)
    return ys[:, :h_dim, :].transpose(2, 0, 1)


# speedup vs baseline: 22.8275x; 1.2926x over previous
import functools

import jax
import jax.numpy as jnp
from jax import lax
from jax.experimental import pallas as pl
from jax.experimental.pallas import tpu as pltpu
from jax.experimental.pallas import tpu_sc as plsc

_NSUB = 16
_LANE = 16
_CHUNK = 128
_CG = 16


def _ih_matmul(x2, w_pad, b_pad, n_tile):
    tb, k = x2.shape
    hp = w_pad.shape[0]
    grid = (hp // n_tile,)

    def body(x_ref, w_ref, b_ref, o_ref):
        acc = lax.dot_general(
            x_ref[...], w_ref[...],
            (((1,), (1,)), ((), ())),
            preferred_element_type=jnp.float32,
        )
        o_ref[...] = acc + b_ref[...]

    return pl.pallas_call(
        body,
        grid=grid,
        in_specs=[
            pl.BlockSpec((tb, k), lambda j: (0, 0)),
            pl.BlockSpec((n_tile, k), lambda j: (j, 0)),
            pl.BlockSpec((1, n_tile), lambda j: (0, j)),
        ],
        out_specs=pl.BlockSpec((tb, n_tile), lambda j: (0, j)),
        out_shape=jax.ShapeDtypeStruct((tb, hp), jnp.float32),
    )(x2, w_pad, b_pad)


def _make_sc_rnn(hp, rows_per_sub, n_chunks, t_steps):
    mesh = plsc.VectorSubcoreMesh(
        core_axis_name="c", subcore_axis_name="s", num_cores=1)
    n_groups = n_chunks // _CG

    def body(ih_ref, cols_ref, rows_ref, vals_ref, zeros_ref,
             ys_ref, h_ref,
             acc, colsv0, colsv1, rowsv0, rowsv1, valsv0, valsv1,
             gath0, gath1, combv, ihv, gsem0, gsem1, ssem):
        colsv = (colsv0, colsv1)
        rowsv = (rowsv0, rowsv1)
        valsv = (valsv0, valsv1)
        gath = (gath0, gath1)
        gsem = (gsem0, gsem1)
        w = lax.axis_index("s")
        slab = pl.ds(w * rows_per_sub, rows_per_sub)

        pltpu.sync_copy(zeros_ref.at[slab], acc.at[slab])
        plsc.subcore_barrier()

        def step(t, _):
            @pl.when(t > 0)
            def phase1():
                def fire(g, b):
                    base = g * _CG
                    pltpu.sync_copy(cols_ref.at[w, pl.ds(base, _CG)], colsv[b])
                    pltpu.sync_copy(rows_ref.at[w, pl.ds(base, _CG)], rowsv[b])
                    pltpu.sync_copy(vals_ref.at[w, pl.ds(base, _CG)], valsv[b])
                    for j in range(_CG):
                        pltpu.async_copy(
                            h_ref.at[colsv[b].at[j]], gath[b].at[j], gsem[b])

                fire(0, 0)

                def outer(i, _):
                    g0 = i * 2
                    for b in range(2):
                        g = g0 + b
                        nxt = g + 1

                        @pl.when(nxt < n_groups)
                        def _():
                            fire(nxt, (b + 1) % 2)

                        for j in range(_CG):
                            pltpu.make_async_copy(
                                h_ref.at[colsv[b].at[j]], gath[b].at[j],
                                gsem[b]).wait()
                        scatters = []
                        for j in range(_CG):
                            def mul16(i2, _):
                                e0 = i2 * _LANE
                                vv = valsv[b][j, pl.ds(e0, _LANE)]
                                for u in range(_LANE):
                                    gath[b][j, e0 + u, :] = (
                                        gath[b][j, e0 + u, :] * vv[u])
                                return 0
                            lax.fori_loop(0, _CHUNK // _LANE, mul16, 0,
                                          unroll=2)
                            scatters.append(pltpu.async_copy(
                                gath[b].at[j], acc.at[rowsv[b].at[j]],
                                ssem, add=True))
                        for c in scatters:
                            c.wait()
                    return 0

                lax.fori_loop(0, n_groups // 2, outer, 0)

            plsc.subcore_barrier()

            pltpu.sync_copy(acc.at[slab], combv)
            pltpu.sync_copy(zeros_ref.at[slab], acc.at[slab])
            pltpu.sync_copy(ih_ref.at[t, slab], ihv)

            def comb(i, _):
                a = combv[i, :] + ihv[i, :]
                e2 = jnp.exp(a * 2.0)
                combv[i, :] = 1.0 - 2.0 / (e2 + 1.0)
                return 0

            lax.fori_loop(0, rows_per_sub, comb, 0, unroll=4)
            pltpu.sync_copy(combv, h_ref.at[slab])
            pltpu.sync_copy(combv, ys_ref.at[t, slab])
            plsc.subcore_barrier()
            return 0

        lax.fori_loop(0, t_steps, step, 0)

    return pl.kernel(
        body,
        out_type=(
            jax.ShapeDtypeStruct((t_steps, hp, _LANE), jnp.float32),
            jax.ShapeDtypeStruct((hp, _LANE), jnp.float32),
        ),
        mesh=mesh,
        scratch_types=[
            pltpu.VMEM_SHARED((hp, _LANE), jnp.float32),
            pltpu.VMEM((_CG, _CHUNK), jnp.int32),
            pltpu.VMEM((_CG, _CHUNK), jnp.int32),
            pltpu.VMEM((_CG, _CHUNK), jnp.int32),
            pltpu.VMEM((_CG, _CHUNK), jnp.int32),
            pltpu.VMEM((_CG, _CHUNK), jnp.float32),
            pltpu.VMEM((_CG, _CHUNK), jnp.float32),
            pltpu.VMEM((_CG, _CHUNK, _LANE), jnp.float32),
            pltpu.VMEM((_CG, _CHUNK, _LANE), jnp.float32),
            pltpu.VMEM((rows_per_sub, _LANE), jnp.float32),
            pltpu.VMEM((rows_per_sub, _LANE), jnp.float32),
            pltpu.SemaphoreType.DMA,
            pltpu.SemaphoreType.DMA,
            pltpu.SemaphoreType.DMA,
        ],
        compiler_params=pltpu.CompilerParams(use_tc_tiling_on_sc=False),
    )


@jax.jit
def kernel(x, hh_indices, hh_values, hh_bias, W_ih):
    b, t, d_in = x.shape
    h_dim = W_ih.shape[0]
    nnz = hh_values.shape[0]
    assert b == _LANE

    n_tile = 2048
    hp = ((h_dim + n_tile - 1) // n_tile) * n_tile
    w_pad = jnp.pad(W_ih, ((0, hp - h_dim), (0, 0)))
    b_pad = jnp.pad(hh_bias.reshape(1, h_dim), ((0, 0), (0, hp - h_dim)))
    x2 = x.swapaxes(0, 1).reshape(t * b, d_in)
    ih2 = _ih_matmul(x2, w_pad, b_pad, n_tile)
    ihs = ih2.reshape(t, b, hp).transpose(0, 2, 1)

    stride = _CHUNK * _CG
    epw = ((nnz // _NSUB + stride - 1) // stride) * stride
    ep = epw * _NSUB
    rows = hh_indices[0]
    cols = hh_indices[1]
    cols_p = jnp.pad(cols, (0, ep - nnz)).reshape(_NSUB, epw // _CHUNK, _CHUNK)
    rows_p = jnp.pad(rows, (0, ep - nnz)).reshape(_NSUB, epw // _CHUNK, _CHUNK)
    vals_p = jnp.pad(hh_values, (0, ep - nnz)).reshape(
        _NSUB, epw // _CHUNK, _CHUNK)

    zeros = jnp.zeros((hp, _LANE), jnp.float32)
    sc_rnn = _make_sc_rnn(hp, hp // _NSUB, epw // _CHUNK, t)
    ys, _ = sc_rnn(ihs, cols_p, rows_p, vals_p, zeros)
    return ys[:, :h_dim, :].transpose(2, 0, 1)
